# baseline (speedup 1.0000x reference)
import functools

import jax
import jax.numpy as jnp
from jax import lax
from jax.experimental import pallas as pl
from jax.experimental.pallas import tpu as pltpu
from jax.experimental.pallas import tpu_sc as plsc

N = 50000
E = 800000
DT_MAX = 0.015

NC = 2
NS = 16

BN = 2000


def _gelu(x):
    return 0.5 * x * (1.0 + lax.erf(x * (2.0 ** -0.5)))


def _node_body(x_ref, wn0, bn0, wn1, bn1, w11, b11, w12, b12,
               w21, b21, w22, b22, w1a, p_ref):
    x = x_ref[...]
    h = _gelu(x @ wn0[...] + bn0[...])
    h = h @ wn1[...] + bn1[...]
    g = (_gelu(h @ w11[...] + b11[...]) @ w12[...] + b12[...]
         + _gelu(h @ w21[...] + b21[...]) @ w22[...] + b22[...])
    p_ref[...] = g @ w1a[...]


GB = 640
NGB = E // GB
GCH = 128
GK = GB // GCH
GIT = -(-NGB // (NC * NS))


def _gather_body(p_hbm, src_hbm, dst_hbm, v_hbm, idxs_v, idxd_v, rows_v,
                 sm0, sm1, sm2, sm3, sm4):
    sems = (sm0, sm1, sm2, sm3, sm4)
    wid = lax.axis_index("s") * NC + lax.axis_index("c")

    def body(k, carry):
        b = k * (NC * NS) + wid

        @pl.when(b < NGB)
        def _():
            pltpu.sync_copy(src_hbm.at[pl.ds(b * GB, GB)], idxs_v)
            pltpu.sync_copy(dst_hbm.at[pl.ds(b * GB, GB)], idxd_v)
            gs = [pltpu.async_copy(p_hbm.at[idxs_v.at[pl.ds(j * GCH, GCH)]],
                                   rows_v.at[pl.ds(j * GCH, GCH)], sems[j])
                  for j in range(GK)]
            ads = []
            for j in range(GK):
                gs[j].wait()
                ads.append(pltpu.async_copy(
                    p_hbm.at[idxd_v.at[pl.ds(j * GCH, GCH)]],
                    rows_v.at[pl.ds(j * GCH, GCH)],
                    sems[j], add=True))
            for cp in ads:
                cp.wait()
            pltpu.sync_copy(rows_v, v_hbm.at[pl.ds(b * GB, GB)])

        return carry

    lax.fori_loop(0, GIT, body, 0)


BE = 2000


def _edge_body(v_ref, ep_ref, w1e, b1e, we2m, bconst, w2m, b2m,
               wh1, bh1, wh2, bh2, sgeo, srow, ag_ref, ap_ref):
    i = pl.program_id(0)
    u = v_ref[...]
    ep = ep_ref[...]
    r = ep[:, 2:3]
    z = u + _gelu(r * w1e[...] + b1e[...]) @ we2m[...] + bconst[...]
    m = _gelu(z) @ w2m[...] + b2m[...]
    hc = _gelu(m @ wh1[...] + bh1[...])
    a = hc @ wh2[...] + bh2[...]
    maskf = (ep[:, 3:4] < ep[:, 4:5]).astype(jnp.float32)
    ag_ref[...] = a + ep @ sgeo[...] + maskf @ srow[...]

    @pl.when(i == 0)
    def _():
        ap_ref[...] = jnp.zeros_like(ap_ref)

    mr = maskf * r
    s0 = jnp.sum(maskf, axis=0, keepdims=True)
    s1 = jnp.sum(mr, axis=0, keepdims=True)
    zrow = jnp.zeros((1, 6), jnp.float32)
    ap_ref[...] += jnp.concatenate([s0, s1, zrow], axis=1)


EPT = E // NS
SCH = 80
SOUT = 25
SIN = EPT // (SOUT * SCH)
ZROWS = N // NS


CH = SIN * SCH


def _scatter_body(idx_hbm, ag_hbm, zeros_hbm, out_hbm,
                  idx_v, ag_v, val_v, stage_v, sem, acc_sh):
    c = lax.axis_index("c")
    s = lax.axis_index("s")
    pltpu.sync_copy(zeros_hbm, stage_v)
    pltpu.sync_copy(stage_v, acc_sh.at[pl.ds(s * ZROWS, ZROWS)])
    plsc.subcore_barrier()
    iot = lax.iota(jnp.int32, 16)
    col = [jnp.full((16,), k, jnp.int32) for k in range(8)]

    def inner(j, _):
        e0 = j * 16
        rows = iot + e0
        flat = rows * 8
        a0 = plsc.load_gather(ag_v, [flat])
        a1 = plsc.load_gather(ag_v, [flat + 1])
        a2 = plsc.load_gather(ag_v, [flat + 2])
        a3 = plsc.load_gather(ag_v, [flat + 3])
        dx = plsc.load_gather(ag_v, [flat + 4])
        dy = plsc.load_gather(ag_v, [flat + 5])
        r = plsc.load_gather(ag_v, [flat + 6])
        mk = plsc.load_gather(ag_v, [flat + 7])
        inv = 1.0 / (r + 1e-12)
        nx = dx * inv
        ny = dy * inv
        mr = mk * r
        n2r = (nx * nx + ny * ny) * mr
        plsc.store_scatter(val_v, [rows, col[0]], a0 * n2r)
        plsc.store_scatter(val_v, [rows, col[1]], a1 * n2r)
        plsc.store_scatter(val_v, [rows, col[2]], (a2 * nx - a3 * ny) * mr)
        plsc.store_scatter(val_v, [rows, col[3]], (a2 * ny + a3 * nx) * mr)
        return _

    def body(o, carry):
        crow = s * (SOUT * SIN) + o * SIN
        ebase = s * EPT + o * CH
        pltpu.sync_copy(idx_hbm.at[c, pl.ds(crow, SIN)], idx_v)
        pltpu.sync_copy(ag_hbm.at[pl.ds(ebase * 8, CH * 8)], ag_v)
        lax.fori_loop(0, CH // 16, inner, 0)
        cps = [pltpu.async_copy(val_v.at[pl.ds(j * SCH, SCH)],
                                acc_sh.at[idx_v.at[j]], sem, add=True)
               for j in range(SIN)]
        for cp in cps:
            cp.wait()
        return carry

    lax.fori_loop(0, SOUT, body, 0)
    plsc.subcore_barrier()
    pltpu.sync_copy(acc_sh.at[pl.ds(s * ZROWS, ZROWS)], stage_v)
    pltpu.sync_copy(stage_v, out_hbm.at[c, pl.ds(s * ZROWS, ZROWS)])


def _final_body(u_ref, d0_ref, d1_ref, sc_ref, out_ref):
    scale = sc_ref[0, 0]
    d = (d0_ref[0] - d1_ref[0]) * scale
    zc = jnp.zeros((d.shape[0], 1), jnp.float32)
    out_ref[...] = u_ref[...] + jnp.concatenate(
        [d[:, 0:2], zc, d[:, 2:4]], axis=1)


def kernel(node_u, edge_index, edge_attr, params):
    f32 = jnp.float32
    src = edge_index[0].astype(jnp.int32)
    dst = edge_index[1].astype(jnp.int32)

    p = params
    (wn0, bn0), (wn1, bn1) = p['phi_node']
    (w11, b11), (w12, b12) = p['phi1']
    (w21, b21), (w22, b22) = p['phi2']
    (we1, be1), (we2, be2) = p['phi_edge']
    (wm1, bm1), (wm2, bm2) = p['phi_msg']
    (wr1, br1), (wr2, br2) = p['psi_rho']
    (wq1, bq1), (wq2, bq2) = p['psi_e']
    (wu1, bu1), (wu2, bu2) = p['psi_rhou']

    w1a = wm1[:64]
    w1b = wm1[64:96]
    we2m = we2 @ w1b
    bconst = (bm1 + be2 @ w1b)[None]
    wh1 = jnp.concatenate([wr1, wq1, wu1], axis=1)
    bh1 = jnp.concatenate([br1, bq1, bu1])[None]
    wh2 = jnp.zeros((192, 8), f32)
    wh2 = wh2.at[0:64, 0:1].set(wr2)
    wh2 = wh2.at[64:128, 1:2].set(wq2)
    wh2 = wh2.at[128:192, 2:4].set(wu2)
    bh2 = jnp.concatenate([br2, bq2, bu2, jnp.zeros((4,), f32)])[None]

    row = lambda b: b[None]

    wspec = lambda shp: pl.BlockSpec(shp, lambda i: (0, 0))
    p_tab = pl.pallas_call(
        _node_body,
        grid=(N // BN,),
        in_specs=[
            pl.BlockSpec((BN, 5), lambda i: (i, 0)),
            wspec((5, 64)), wspec((1, 64)), wspec((64, 64)), wspec((1, 64)),
            wspec((64, 32)), wspec((1, 32)), wspec((32, 64)), wspec((1, 64)),
            wspec((64, 32)), wspec((1, 32)), wspec((32, 64)), wspec((1, 64)),
            wspec((64, 128)),
        ],
        out_specs=pl.BlockSpec((BN, 128), lambda i: (i, 0)),
        out_shape=jax.ShapeDtypeStruct((N, 128), f32),
    )(node_u, wn0, row(bn0), wn1, row(bn1),
      w11, row(b11), w12, row(b12), w21, row(b21), w22, row(b22), w1a)

    mesh = plsc.VectorSubcoreMesh(core_axis_name="c", subcore_axis_name="s")
    gather_params = pltpu.CompilerParams(use_tc_tiling_on_sc=True)
    sc_params = pltpu.CompilerParams(use_tc_tiling_on_sc=False,
                                     needs_layout_passes=False)
    v = pl.kernel(
        _gather_body,
        mesh=mesh,
        compiler_params=gather_params,
        out_type=jax.ShapeDtypeStruct((E, 128), f32),
        scratch_types=[
            pltpu.VMEM((GB,), jnp.int32),
            pltpu.VMEM((GB,), jnp.int32),
            pltpu.VMEM((GB, 128), f32),
        ] + [pltpu.SemaphoreType.DMA] * GK,
    )(p_tab, src, dst)

    srcf = src.astype(f32)[:, None]
    dstf = dst.astype(f32)[:, None]
    eprep = jnp.concatenate([edge_attr, srcf, dstf], axis=1)
    sgeo = jnp.zeros((5, 8), f32).at[0, 4].set(1.0).at[1, 5].set(1.0) \
        .at[2, 6].set(1.0)
    srow = jnp.zeros((1, 8), f32).at[0, 7].set(1.0)
    ag, ap = pl.pallas_call(
        _edge_body,
        grid=(E // BE,),
        in_specs=[
            pl.BlockSpec((BE, 128), lambda i: (i, 0)),
            pl.BlockSpec((BE, 5), lambda i: (i, 0)),
            wspec((1, 32)), wspec((1, 32)),
            wspec((32, 128)), wspec((1, 128)), wspec((128, 64)), wspec((1, 64)),
            wspec((64, 192)), wspec((1, 192)), wspec((192, 8)), wspec((1, 8)),
            wspec((5, 8)), wspec((1, 8)),
        ],
        out_specs=[
            pl.BlockSpec((BE, 8), lambda i: (i, 0)),
            pl.BlockSpec((1, 8), lambda i: (0, 0)),
        ],
        out_shape=[
            jax.ShapeDtypeStruct((E, 8), f32),
            jax.ShapeDtypeStruct((1, 8), f32),
        ],
    )(v, eprep, we1, row(be1), we2m, bconst, wm2, row(bm2),
      wh1, bh1, wh2, bh2, sgeo, srow)

    idx3 = edge_index.astype(jnp.int32).reshape(2, E // SCH, SCH)
    zeros_tile = jnp.zeros((ZROWS, 4), f32)
    P = pl.kernel(
        _scatter_body,
        mesh=mesh,
        compiler_params=sc_params,
        out_type=jax.ShapeDtypeStruct((2, N, 4), f32),
        scratch_types=[
            pltpu.VMEM((SIN, SCH), jnp.int32),
            pltpu.VMEM((CH * 8,), f32),
            pltpu.VMEM((CH, 4), f32),
            pltpu.VMEM((ZROWS, 4), f32),
            pltpu.SemaphoreType.DMA,
            pltpu.VMEM_SHARED((N, 4), f32),
        ],
    )(idx3, ag.reshape(-1), zeros_tile)

    dt = DT_MAX * jax.nn.sigmoid(p['s'])
    area = (ap[0, 1] / ap[0, 0]) ** 2
    scale = jnp.reshape(-dt / area, (1, 1)).astype(f32)

    out = pl.pallas_call(
        _final_body,
        grid=(N // BN,),
        in_specs=[
            pl.BlockSpec((BN, 5), lambda i: (i, 0)),
            pl.BlockSpec((1, BN, 4), lambda i: (0, i, 0)),
            pl.BlockSpec((1, BN, 4), lambda i: (1, i, 0)),
            pl.BlockSpec((1, 1), lambda i: (0, 0)),
        ],
        out_specs=pl.BlockSpec((BN, 5), lambda i: (i, 0)),
        out_shape=jax.ShapeDtypeStruct((N, 5), f32),
    )(node_u, P, P, scale)
    return out


# speedup vs baseline: 1.0508x; 1.0508x over previous
import functools

import jax
import jax.numpy as jnp
from jax import lax
from jax.experimental import pallas as pl
from jax.experimental.pallas import tpu as pltpu
from jax.experimental.pallas import tpu_sc as plsc

N = 50000
E = 800000
DT_MAX = 0.015

NC = 2
NS = 16

BN = 2000


def _gelu(x):
    return 0.5 * x * (1.0 + lax.erf(x * (2.0 ** -0.5)))


def _node_body(x_ref, wn0, bn0, wn1, bn1, w11, b11, w12, b12,
               w21, b21, w22, b22, w1a, p_ref):
    x = x_ref[...]
    h = _gelu(x @ wn0[...] + bn0[...])
    h = h @ wn1[...] + bn1[...]
    g = (_gelu(h @ w11[...] + b11[...]) @ w12[...] + b12[...]
         + _gelu(h @ w21[...] + b21[...]) @ w22[...] + b22[...])
    p_ref[...] = g @ w1a[...]


GB = 640
NGB = E // GB
GCH = 128
GK = GB // GCH
GIT = -(-NGB // (NC * NS))


def _gather_body(p_hbm, src_hbm, dst_hbm, v_hbm, idxs_v, idxd_v, rows_v,
                 sm0, sm1, sm2, sm3, sm4):
    sems = (sm0, sm1, sm2, sm3, sm4)
    wid = lax.axis_index("s") * NC + lax.axis_index("c")

    def body(k, carry):
        b = k * (NC * NS) + wid

        @pl.when(b < NGB)
        def _():
            pltpu.sync_copy(src_hbm.at[pl.ds(b * GB, GB)], idxs_v)
            pltpu.sync_copy(dst_hbm.at[pl.ds(b * GB, GB)], idxd_v)
            gs = [pltpu.async_copy(p_hbm.at[idxs_v.at[pl.ds(j * GCH, GCH)]],
                                   rows_v.at[pl.ds(j * GCH, GCH)], sems[j])
                  for j in range(GK)]
            ads = []
            for j in range(GK):
                gs[j].wait()
                ads.append(pltpu.async_copy(
                    p_hbm.at[idxd_v.at[pl.ds(j * GCH, GCH)]],
                    rows_v.at[pl.ds(j * GCH, GCH)],
                    sems[j], add=True))
            for cp in ads:
                cp.wait()
            pltpu.sync_copy(rows_v, v_hbm.at[pl.ds(b * GB, GB)])

        return carry

    lax.fori_loop(0, GIT, body, 0)


BE = 8000


def _edge_body(v_ref, ep_ref, w1e, b1e, we2m, bconst, w2m, b2m,
               wh1, bh1, wh2, bh2, sgeo, srow, ag_ref, ap_ref):
    i = pl.program_id(0)
    u = v_ref[...]
    ep = ep_ref[...]
    r = ep[:, 2:3]
    z = u + _gelu(r * w1e[...] + b1e[...]) @ we2m[...] + bconst[...]
    m = _gelu(z) @ w2m[...] + b2m[...]
    hc = _gelu(m @ wh1[...] + bh1[...])
    a = hc @ wh2[...] + bh2[...]
    maskf = (ep[:, 3:4] < ep[:, 4:5]).astype(jnp.float32)
    ag_ref[...] = a + ep @ sgeo[...] + maskf @ srow[...]

    @pl.when(i == 0)
    def _():
        ap_ref[...] = jnp.zeros_like(ap_ref)

    mr = maskf * r
    s0 = jnp.sum(maskf, axis=0, keepdims=True)
    s1 = jnp.sum(mr, axis=0, keepdims=True)
    zrow = jnp.zeros((1, 6), jnp.float32)
    ap_ref[...] += jnp.concatenate([s0, s1, zrow], axis=1)


EPT = E // NS
SCH = 80
SOUT = 25
SIN = EPT // (SOUT * SCH)
ZROWS = N // NS


CH = SIN * SCH


def _scatter_body(idx_hbm, ag_hbm, zeros_hbm, out_hbm,
                  idx_v, ag_v, val_v, stage_v, sem, acc_sh):
    c = lax.axis_index("c")
    s = lax.axis_index("s")
    pltpu.sync_copy(zeros_hbm, stage_v)
    pltpu.sync_copy(stage_v, acc_sh.at[pl.ds(s * ZROWS, ZROWS)])
    pltpu.sync_copy(zeros_hbm.at[pl.ds(0, CH)], val_v)
    plsc.subcore_barrier()
    iot = lax.iota(jnp.int32, 16)
    col = [jnp.full((16,), k, jnp.int32) for k in range(8)]

    def inner(j, _):
        e0 = j * 16
        rows = iot + e0
        flat = rows * 8
        a0 = plsc.load_gather(ag_v, [flat])
        a1 = plsc.load_gather(ag_v, [flat + 1])
        a2 = plsc.load_gather(ag_v, [flat + 2])
        a3 = plsc.load_gather(ag_v, [flat + 3])
        dx = plsc.load_gather(ag_v, [flat + 4])
        dy = plsc.load_gather(ag_v, [flat + 5])
        r = plsc.load_gather(ag_v, [flat + 6])
        mk = plsc.load_gather(ag_v, [flat + 7])
        inv = 1.0 / (r + 1e-12)
        nx = dx * inv
        ny = dy * inv
        mr = mk * r
        n2r = (nx * nx + ny * ny) * mr
        plsc.store_scatter(val_v, [rows, col[0]], a0 * n2r)
        plsc.store_scatter(val_v, [rows, col[1]], a1 * n2r)
        plsc.store_scatter(val_v, [rows, col[3]], (a2 * nx - a3 * ny) * mr)
        plsc.store_scatter(val_v, [rows, col[4]], (a2 * ny + a3 * nx) * mr)
        return _

    def body(o, carry):
        crow = s * (SOUT * SIN) + o * SIN
        ebase = s * EPT + o * CH
        pltpu.sync_copy(idx_hbm.at[c, pl.ds(crow, SIN)], idx_v)
        pltpu.sync_copy(ag_hbm.at[pl.ds(ebase * 8, CH * 8)], ag_v)
        lax.fori_loop(0, CH // 16, inner, 0)
        cps = [pltpu.async_copy(val_v.at[pl.ds(j * SCH, SCH)],
                                acc_sh.at[idx_v.at[j]], sem, add=True)
               for j in range(SIN)]
        for cp in cps:
            cp.wait()
        return carry

    lax.fori_loop(0, SOUT, body, 0)
    plsc.subcore_barrier()
    pltpu.sync_copy(acc_sh.at[pl.ds(s * ZROWS, ZROWS)], stage_v)
    pltpu.sync_copy(stage_v, out_hbm.at[c, pl.ds(s * ZROWS, ZROWS)])


def _final_body(u_ref, d0_ref, d1_ref, sc_ref, out_ref):
    scale = sc_ref[0, 0]
    d = (d0_ref[0] - d1_ref[0]) * scale
    out_ref[...] = u_ref[...] + d[:, 0:5]


def kernel(node_u, edge_index, edge_attr, params):
    f32 = jnp.float32
    src = edge_index[0].astype(jnp.int32)
    dst = edge_index[1].astype(jnp.int32)

    p = params
    (wn0, bn0), (wn1, bn1) = p['phi_node']
    (w11, b11), (w12, b12) = p['phi1']
    (w21, b21), (w22, b22) = p['phi2']
    (we1, be1), (we2, be2) = p['phi_edge']
    (wm1, bm1), (wm2, bm2) = p['phi_msg']
    (wr1, br1), (wr2, br2) = p['psi_rho']
    (wq1, bq1), (wq2, bq2) = p['psi_e']
    (wu1, bu1), (wu2, bu2) = p['psi_rhou']

    w1a = wm1[:64]
    w1b = wm1[64:96]
    we2m = we2 @ w1b
    bconst = (bm1 + be2 @ w1b)[None]
    wh1 = jnp.concatenate([wr1, wq1, wu1], axis=1)
    bh1 = jnp.concatenate([br1, bq1, bu1])[None]
    wh2 = jnp.zeros((192, 8), f32)
    wh2 = wh2.at[0:64, 0:1].set(wr2)
    wh2 = wh2.at[64:128, 1:2].set(wq2)
    wh2 = wh2.at[128:192, 2:4].set(wu2)
    bh2 = jnp.concatenate([br2, bq2, bu2, jnp.zeros((4,), f32)])[None]

    row = lambda b: b[None]

    wspec = lambda shp: pl.BlockSpec(shp, lambda i: (0, 0))
    p_tab = pl.pallas_call(
        _node_body,
        grid=(N // BN,),
        in_specs=[
            pl.BlockSpec((BN, 5), lambda i: (i, 0)),
            wspec((5, 64)), wspec((1, 64)), wspec((64, 64)), wspec((1, 64)),
            wspec((64, 32)), wspec((1, 32)), wspec((32, 64)), wspec((1, 64)),
            wspec((64, 32)), wspec((1, 32)), wspec((32, 64)), wspec((1, 64)),
            wspec((64, 128)),
        ],
        out_specs=pl.BlockSpec((BN, 128), lambda i: (i, 0)),
        out_shape=jax.ShapeDtypeStruct((N, 128), f32),
    )(node_u, wn0, row(bn0), wn1, row(bn1),
      w11, row(b11), w12, row(b12), w21, row(b21), w22, row(b22), w1a)

    mesh = plsc.VectorSubcoreMesh(core_axis_name="c", subcore_axis_name="s")
    gather_params = pltpu.CompilerParams(use_tc_tiling_on_sc=True)
    sc_params = pltpu.CompilerParams(use_tc_tiling_on_sc=False,
                                     needs_layout_passes=False)
    v = pl.kernel(
        _gather_body,
        mesh=mesh,
        compiler_params=gather_params,
        out_type=jax.ShapeDtypeStruct((E, 128), f32),
        scratch_types=[
            pltpu.VMEM((GB,), jnp.int32),
            pltpu.VMEM((GB,), jnp.int32),
            pltpu.VMEM((GB, 128), f32),
        ] + [pltpu.SemaphoreType.DMA] * GK,
    )(p_tab, src, dst)

    srcf = src.astype(f32)[:, None]
    dstf = dst.astype(f32)[:, None]
    eprep = jnp.concatenate([edge_attr, srcf, dstf], axis=1)
    sgeo = jnp.zeros((5, 8), f32).at[0, 4].set(1.0).at[1, 5].set(1.0) \
        .at[2, 6].set(1.0)
    srow = jnp.zeros((1, 8), f32).at[0, 7].set(1.0)
    ag, ap = pl.pallas_call(
        _edge_body,
        grid=(E // BE,),
        in_specs=[
            pl.BlockSpec((BE, 128), lambda i: (i, 0)),
            pl.BlockSpec((BE, 5), lambda i: (i, 0)),
            wspec((1, 32)), wspec((1, 32)),
            wspec((32, 128)), wspec((1, 128)), wspec((128, 64)), wspec((1, 64)),
            wspec((64, 192)), wspec((1, 192)), wspec((192, 8)), wspec((1, 8)),
            wspec((5, 8)), wspec((1, 8)),
        ],
        out_specs=[
            pl.BlockSpec((BE, 8), lambda i: (i, 0)),
            pl.BlockSpec((1, 8), lambda i: (0, 0)),
        ],
        out_shape=[
            jax.ShapeDtypeStruct((E, 8), f32),
            jax.ShapeDtypeStruct((1, 8), f32),
        ],
    )(v, eprep, we1, row(be1), we2m, bconst, wm2, row(bm2),
      wh1, bh1, wh2, bh2, sgeo, srow)

    idx3 = edge_index.astype(jnp.int32).reshape(2, E // SCH, SCH)
    zeros_tile = jnp.zeros((ZROWS, 8), f32)
    P = pl.kernel(
        _scatter_body,
        mesh=mesh,
        compiler_params=sc_params,
        out_type=jax.ShapeDtypeStruct((2, N, 8), f32),
        scratch_types=[
            pltpu.VMEM((SIN, SCH), jnp.int32),
            pltpu.VMEM((CH * 8,), f32),
            pltpu.VMEM((CH, 8), f32),
            pltpu.VMEM((ZROWS, 8), f32),
            pltpu.SemaphoreType.DMA,
            pltpu.VMEM_SHARED((N, 8), f32),
        ],
    )(idx3, ag.reshape(-1), zeros_tile)

    dt = DT_MAX * jax.nn.sigmoid(p['s'])
    area = (ap[0, 1] / ap[0, 0]) ** 2
    scale = jnp.reshape(-dt / area, (1, 1)).astype(f32)

    out = pl.pallas_call(
        _final_body,
        grid=(N // BN,),
        in_specs=[
            pl.BlockSpec((BN, 5), lambda i: (i, 0)),
            pl.BlockSpec((1, BN, 8), lambda i: (0, i, 0)),
            pl.BlockSpec((1, BN, 8), lambda i: (1, i, 0)),
            pl.BlockSpec((1, 1), lambda i: (0, 0)),
        ],
        out_specs=pl.BlockSpec((BN, 5), lambda i: (i, 0)),
        out_shape=jax.ShapeDtypeStruct((N, 5), f32),
    )(node_u, P, P, scale)
    return out


# speedup vs baseline: 1.0549x; 1.0038x over previous
import functools

import jax
import jax.numpy as jnp
from jax import lax
from jax.experimental import pallas as pl
from jax.experimental.pallas import tpu as pltpu
from jax.experimental.pallas import tpu_sc as plsc

N = 50000
E = 800000
DT_MAX = 0.015

NC = 2
NS = 16

BN = 2000


def _gelu(x):
    return 0.5 * x * (1.0 + lax.erf(x * (2.0 ** -0.5)))


def _node_body(x_ref, wn0, bn0, wn1, bn1, w11, b11, w12, b12,
               w21, b21, w22, b22, w1a, p_ref):
    x = x_ref[...]
    h = _gelu(x @ wn0[...] + bn0[...])
    h = h @ wn1[...] + bn1[...]
    g = (_gelu(h @ w11[...] + b11[...]) @ w12[...] + b12[...]
         + _gelu(h @ w21[...] + b21[...]) @ w22[...] + b22[...])
    p_ref[...] = g @ w1a[...]


GB = 640
NGB = E // GB
GCH = 128
GK = GB // GCH
GIT = -(-NGB // (NC * NS))


def _gather_body(p_hbm, src_hbm, dst_hbm, v_hbm, idxs_v, idxd_v, rows_v,
                 sm0, sm1, sm2, sm3, sm4):
    sems = (sm0, sm1, sm2, sm3, sm4)
    wid = lax.axis_index("s") * NC + lax.axis_index("c")

    def body(k, carry):
        b = k * (NC * NS) + wid

        @pl.when(b < NGB)
        def _():
            pltpu.sync_copy(src_hbm.at[pl.ds(b * GB, GB)], idxs_v)
            pltpu.sync_copy(dst_hbm.at[pl.ds(b * GB, GB)], idxd_v)
            gs = [pltpu.async_copy(p_hbm.at[idxs_v.at[pl.ds(j * GCH, GCH)]],
                                   rows_v.at[pl.ds(j * GCH, GCH)], sems[j])
                  for j in range(GK)]
            ads = []
            for j in range(GK):
                gs[j].wait()
                ads.append(pltpu.async_copy(
                    p_hbm.at[idxd_v.at[pl.ds(j * GCH, GCH)]],
                    rows_v.at[pl.ds(j * GCH, GCH)],
                    sems[j], add=True))
            for cp in ads:
                cp.wait()
            pltpu.sync_copy(rows_v, v_hbm.at[pl.ds(b * GB, GB)])

        return carry

    lax.fori_loop(0, GIT, body, 0)


BE = 8000


def _edge_body(v_ref, ep_ref, w1e, b1e, we2m, bconst, w2m, b2m,
               wh1, bh1, wh2, bh2, sgeo, srow, ag_ref, ap_ref):
    i = pl.program_id(0)
    u = v_ref[...]
    ep = ep_ref[...]
    r = ep[:, 2:3]
    z = u + _gelu(r * w1e[...] + b1e[...]) @ we2m[...] + bconst[...]
    m = _gelu(z) @ w2m[...] + b2m[...]
    hc = _gelu(m @ wh1[...] + bh1[...])
    a = hc @ wh2[...] + bh2[...]
    maskf = (ep[:, 3:4] < ep[:, 4:5]).astype(jnp.float32)
    ag_ref[...] = a + ep @ sgeo[...] + maskf @ srow[...]

    @pl.when(i == 0)
    def _():
        ap_ref[...] = jnp.zeros_like(ap_ref)

    mr = maskf * r
    s0 = jnp.sum(maskf, axis=0, keepdims=True)
    s1 = jnp.sum(mr, axis=0, keepdims=True)
    zrow = jnp.zeros((1, 6), jnp.float32)
    ap_ref[...] += jnp.concatenate([s0, s1, zrow], axis=1)


EPT = E // NS
SCH = 80
SOUT = 25
SIN = EPT // (SOUT * SCH)
ZROWS = N // NS


CH = SIN * SCH


def _scatter_body(idx_hbm, ag_hbm, zeros_hbm, out_hbm,
                  idxA, idxB, ag_v, valA, valB, stage_v, semA, semB, acc_sh):
    c = lax.axis_index("c")
    s = lax.axis_index("s")
    pltpu.sync_copy(zeros_hbm, stage_v)
    pltpu.sync_copy(stage_v, acc_sh.at[pl.ds(s * ZROWS, ZROWS)])
    pltpu.sync_copy(zeros_hbm.at[pl.ds(0, CH)], valA)
    pltpu.sync_copy(zeros_hbm.at[pl.ds(0, CH)], valB)
    plsc.subcore_barrier()
    iot = lax.iota(jnp.int32, 16)
    col = [jnp.full((16,), k, jnp.int32) for k in range(8)]

    def make_inner(val_v):
        def inner(j, _):
            e0 = j * 16
            rows = iot + e0
            flat = rows * 8
            a0 = plsc.load_gather(ag_v, [flat])
            a1 = plsc.load_gather(ag_v, [flat + 1])
            a2 = plsc.load_gather(ag_v, [flat + 2])
            a3 = plsc.load_gather(ag_v, [flat + 3])
            dx = plsc.load_gather(ag_v, [flat + 4])
            dy = plsc.load_gather(ag_v, [flat + 5])
            r = plsc.load_gather(ag_v, [flat + 6])
            mk = plsc.load_gather(ag_v, [flat + 7])
            inv = 1.0 / (r + 1e-12)
            nx = dx * inv
            ny = dy * inv
            mr = mk * r
            n2r = (nx * nx + ny * ny) * mr
            plsc.store_scatter(val_v, [rows, col[0]], a0 * n2r)
            plsc.store_scatter(val_v, [rows, col[1]], a1 * n2r)
            plsc.store_scatter(val_v, [rows, col[3]], (a2 * nx - a3 * ny) * mr)
            plsc.store_scatter(val_v, [rows, col[4]], (a2 * ny + a3 * nx) * mr)
            return _
        return inner

    def drain(val_v, sem):
        for j in range(SIN):
            pltpu.make_async_copy(zeros_hbm.at[pl.ds(0, SCH)],
                                  val_v.at[pl.ds(j * SCH, SCH)], sem).wait()

    def sub(o, q, idx_v, val_v, sem):
        @pl.when(q > 0)
        def _():
            drain(val_v, sem)

        @pl.when(o < SOUT)
        def _():
            crow = s * (SOUT * SIN) + o * SIN
            ebase = s * EPT + o * CH
            pltpu.sync_copy(idx_hbm.at[c, pl.ds(crow, SIN)], idx_v)
            pltpu.sync_copy(ag_hbm.at[pl.ds(ebase * 8, CH * 8)], ag_v)
            lax.fori_loop(0, CH // 16, make_inner(val_v), 0)
            for j in range(SIN):
                pltpu.async_copy(val_v.at[pl.ds(j * SCH, SCH)],
                                 acc_sh.at[idx_v.at[j]], sem, add=True)

    def body(q, carry):
        sub(2 * q, q, idxA, valA, semA)
        sub(2 * q + 1, q, idxB, valB, semB)
        return carry

    lax.fori_loop(0, (SOUT + 1) // 2, body, 0)
    drain(valA, semA)
    plsc.subcore_barrier()
    pltpu.sync_copy(acc_sh.at[pl.ds(s * ZROWS, ZROWS)], stage_v)
    pltpu.sync_copy(stage_v, out_hbm.at[c, pl.ds(s * ZROWS, ZROWS)])


def _final_body(u_ref, d0_ref, d1_ref, sc_ref, out_ref):
    scale = sc_ref[0, 0]
    d = (d0_ref[0] - d1_ref[0]) * scale
    out_ref[...] = u_ref[...] + d[:, 0:5]


def kernel(node_u, edge_index, edge_attr, params):
    f32 = jnp.float32
    src = edge_index[0].astype(jnp.int32)
    dst = edge_index[1].astype(jnp.int32)

    p = params
    (wn0, bn0), (wn1, bn1) = p['phi_node']
    (w11, b11), (w12, b12) = p['phi1']
    (w21, b21), (w22, b22) = p['phi2']
    (we1, be1), (we2, be2) = p['phi_edge']
    (wm1, bm1), (wm2, bm2) = p['phi_msg']
    (wr1, br1), (wr2, br2) = p['psi_rho']
    (wq1, bq1), (wq2, bq2) = p['psi_e']
    (wu1, bu1), (wu2, bu2) = p['psi_rhou']

    w1a = wm1[:64]
    w1b = wm1[64:96]
    we2m = we2 @ w1b
    bconst = (bm1 + be2 @ w1b)[None]
    wh1 = jnp.concatenate([wr1, wq1, wu1], axis=1)
    bh1 = jnp.concatenate([br1, bq1, bu1])[None]
    wh2 = jnp.zeros((192, 8), f32)
    wh2 = wh2.at[0:64, 0:1].set(wr2)
    wh2 = wh2.at[64:128, 1:2].set(wq2)
    wh2 = wh2.at[128:192, 2:4].set(wu2)
    bh2 = jnp.concatenate([br2, bq2, bu2, jnp.zeros((4,), f32)])[None]

    row = lambda b: b[None]

    wspec = lambda shp: pl.BlockSpec(shp, lambda i: (0, 0))
    p_tab = pl.pallas_call(
        _node_body,
        grid=(N // BN,),
        in_specs=[
            pl.BlockSpec((BN, 5), lambda i: (i, 0)),
            wspec((5, 64)), wspec((1, 64)), wspec((64, 64)), wspec((1, 64)),
            wspec((64, 32)), wspec((1, 32)), wspec((32, 64)), wspec((1, 64)),
            wspec((64, 32)), wspec((1, 32)), wspec((32, 64)), wspec((1, 64)),
            wspec((64, 128)),
        ],
        out_specs=pl.BlockSpec((BN, 128), lambda i: (i, 0)),
        out_shape=jax.ShapeDtypeStruct((N, 128), f32),
    )(node_u, wn0, row(bn0), wn1, row(bn1),
      w11, row(b11), w12, row(b12), w21, row(b21), w22, row(b22), w1a)

    mesh = plsc.VectorSubcoreMesh(core_axis_name="c", subcore_axis_name="s")
    gather_params = pltpu.CompilerParams(use_tc_tiling_on_sc=True)
    sc_params = pltpu.CompilerParams(use_tc_tiling_on_sc=False,
                                     needs_layout_passes=False)
    v = pl.kernel(
        _gather_body,
        mesh=mesh,
        compiler_params=gather_params,
        out_type=jax.ShapeDtypeStruct((E, 128), f32),
        scratch_types=[
            pltpu.VMEM((GB,), jnp.int32),
            pltpu.VMEM((GB,), jnp.int32),
            pltpu.VMEM((GB, 128), f32),
        ] + [pltpu.SemaphoreType.DMA] * GK,
    )(p_tab, src, dst)

    srcf = src.astype(f32)[:, None]
    dstf = dst.astype(f32)[:, None]
    eprep = jnp.concatenate([edge_attr, srcf, dstf], axis=1)
    sgeo = jnp.zeros((5, 8), f32).at[0, 4].set(1.0).at[1, 5].set(1.0) \
        .at[2, 6].set(1.0)
    srow = jnp.zeros((1, 8), f32).at[0, 7].set(1.0)
    ag, ap = pl.pallas_call(
        _edge_body,
        grid=(E // BE,),
        in_specs=[
            pl.BlockSpec((BE, 128), lambda i: (i, 0)),
            pl.BlockSpec((BE, 5), lambda i: (i, 0)),
            wspec((1, 32)), wspec((1, 32)),
            wspec((32, 128)), wspec((1, 128)), wspec((128, 64)), wspec((1, 64)),
            wspec((64, 192)), wspec((1, 192)), wspec((192, 8)), wspec((1, 8)),
            wspec((5, 8)), wspec((1, 8)),
        ],
        out_specs=[
            pl.BlockSpec((BE, 8), lambda i: (i, 0)),
            pl.BlockSpec((1, 8), lambda i: (0, 0)),
        ],
        out_shape=[
            jax.ShapeDtypeStruct((E, 8), f32),
            jax.ShapeDtypeStruct((1, 8), f32),
        ],
    )(v, eprep, we1, row(be1), we2m, bconst, wm2, row(bm2),
      wh1, bh1, wh2, bh2, sgeo, srow)

    idx3 = edge_index.astype(jnp.int32).reshape(2, E // SCH, SCH)
    zeros_tile = jnp.zeros((ZROWS, 8), f32)
    P = pl.kernel(
        _scatter_body,
        mesh=mesh,
        compiler_params=sc_params,
        out_type=jax.ShapeDtypeStruct((2, N, 8), f32),
        scratch_types=[
            pltpu.VMEM((SIN, SCH), jnp.int32),
            pltpu.VMEM((SIN, SCH), jnp.int32),
            pltpu.VMEM((CH * 8,), f32),
            pltpu.VMEM((CH, 8), f32),
            pltpu.VMEM((CH, 8), f32),
            pltpu.VMEM((ZROWS, 8), f32),
            pltpu.SemaphoreType.DMA,
            pltpu.SemaphoreType.DMA,
            pltpu.VMEM_SHARED((N, 8), f32),
        ],
    )(idx3, ag.reshape(-1), zeros_tile)

    dt = DT_MAX * jax.nn.sigmoid(p['s'])
    area = (ap[0, 1] / ap[0, 0]) ** 2
    scale = jnp.reshape(-dt / area, (1, 1)).astype(f32)

    out = pl.pallas_call(
        _final_body,
        grid=(N // BN,),
        in_specs=[
            pl.BlockSpec((BN, 5), lambda i: (i, 0)),
            pl.BlockSpec((1, BN, 8), lambda i: (0, i, 0)),
            pl.BlockSpec((1, BN, 8), lambda i: (1, i, 0)),
            pl.BlockSpec((1, 1), lambda i: (0, 0)),
        ],
        out_specs=pl.BlockSpec((BN, 5), lambda i: (i, 0)),
        out_shape=jax.ShapeDtypeStruct((N, 5), f32),
    )(node_u, P, P, scale)
    return out
